# baseline (speedup 1.0000x reference)
import jax
import jax.numpy as jnp
from jax import lax
from jax.experimental import pallas as pl

_E = 64
_GROUPS = 8
_GROUP_SIZE = 8
_TOPK_GROUPS = 4
_TOP_K = 8
_SCALE = 2.5
_TB = 512


def _router_body(x_ref, w_ref, b_ref, wout_ref, iout_ref):
    x = x_ref[...]
    w = w_ref[...]
    scores = jnp.dot(x, w, preferred_element_type=jnp.float32)
    s = jax.nn.sigmoid(scores)
    sb = s + b_ref[...]

    tb = s.shape[0]
    iota_e = lax.broadcasted_iota(jnp.int32, (tb, _E), 1)
    gid = iota_e // _GROUP_SIZE
    neg_inf = jnp.float32(-jnp.inf)

    gcols = []
    for g in range(_GROUPS):
        vg = jnp.where(gid == g, sb, neg_inf)
        m1 = jnp.max(vg, axis=-1, keepdims=True)
        i1 = jnp.min(jnp.where(vg == m1, iota_e, _E), axis=-1, keepdims=True)
        m2 = jnp.max(jnp.where(iota_e == i1, neg_inf, vg), axis=-1, keepdims=True)
        gcols.append(m1 + m2)
    gsc = jnp.concatenate(gcols, axis=-1)

    iota_g = lax.broadcasted_iota(jnp.int32, (tb, _GROUPS), 1)
    gmask = jnp.zeros((tb, _E), jnp.bool_)
    for _ in range(_TOPK_GROUPS):
        m = jnp.max(gsc, axis=-1, keepdims=True)
        gi = jnp.min(jnp.where(gsc == m, iota_g, _GROUPS), axis=-1, keepdims=True)
        gmask = jnp.logical_or(gmask, gid == gi)
        gsc = jnp.where(iota_g == gi, neg_inf, gsc)

    sm = jnp.where(gmask, sb, 0.0)
    idx_cols = []
    w_cols = []
    for _ in range(_TOP_K):
        m = jnp.max(sm, axis=-1, keepdims=True)
        i = jnp.min(jnp.where(sm == m, iota_e, _E), axis=-1, keepdims=True)
        onehot = iota_e == i
        w_cols.append(jnp.sum(jnp.where(onehot, s, 0.0), axis=-1, keepdims=True))
        idx_cols.append(i)
        sm = jnp.where(onehot, neg_inf, sm)
    wts = jnp.concatenate(w_cols, axis=-1)
    inds = jnp.concatenate(idx_cols, axis=-1)

    wts = wts / (jnp.sum(wts, axis=-1, keepdims=True) + 1e-20) * _SCALE
    wout_ref[...] = wts
    iout_ref[...] = inds


def kernel(x_TD, kernel_DE, bias_E):
    x_TD = jnp.asarray(x_TD, jnp.float32)
    t, d = x_TD.shape
    bias = bias_E.reshape(1, _E).astype(jnp.float32)
    grid = t // _TB
    weights, indices = pl.pallas_call(
        _router_body,
        grid=(grid,),
        in_specs=[
            pl.BlockSpec((_TB, d), lambda i: (i, 0)),
            pl.BlockSpec((d, _E), lambda i: (0, 0)),
            pl.BlockSpec((1, _E), lambda i: (0, 0)),
        ],
        out_specs=[
            pl.BlockSpec((_TB, _TOP_K), lambda i: (i, 0)),
            pl.BlockSpec((_TB, _TOP_K), lambda i: (i, 0)),
        ],
        out_shape=[
            jax.ShapeDtypeStruct((t, _TOP_K), jnp.float32),
            jax.ShapeDtypeStruct((t, _TOP_K), jnp.int32),
        ],
    )(x_TD, kernel_DE.astype(jnp.float32), bias)
    return (weights, indices)


# speedup vs baseline: 2.5249x; 2.5249x over previous
import jax
import jax.numpy as jnp
from jax import lax
from jax.experimental import pallas as pl

_E = 64
_GROUPS = 8
_GROUP_SIZE = 8
_TOPK_GROUPS = 4
_TOP_K = 8
_SCALE = 2.5
_TB = 512

_IMIN = -(2 ** 31)
_FIX = float(2 ** 23)


def _router_body(x_ref, wt_ref, b_ref, wout_ref, iout_ref):
    x = x_ref[...]
    wt = wt_ref[...]
    scores = lax.dot_general(
        wt, x, (((1,), (1,)), ((), ())), preferred_element_type=jnp.float32
    )
    tb = scores.shape[-1]
    s3 = jax.nn.sigmoid(scores).reshape(_GROUPS, _GROUP_SIZE, tb)
    sb3 = s3 + b_ref[...][:, :, None]

    gio = lax.broadcasted_iota(jnp.int32, (_GROUPS, _GROUP_SIZE, tb), 0)
    mio = lax.broadcasted_iota(jnp.int32, (_GROUPS, _GROUP_SIZE, tb), 1)
    eidx = gio * _GROUP_SIZE + mio
    zkb = 63 - eidx

    q = jnp.clip(sb3, -3.9, 3.9) * _FIX
    kb = (q.astype(jnp.int32) << 6) | zkb

    m1 = jnp.max(kb, axis=1, keepdims=True)
    m2 = jnp.max(jnp.where(kb == m1, _IMIN, kb), axis=1, keepdims=True)
    gq = (m1 >> 6) + (m2 >> 6)

    giota = gio[:, :1, :]
    gk = (gq << 3) | (7 - giota)
    gmask = jnp.zeros((_GROUPS, 1, tb), jnp.bool_)
    for _ in range(_TOPK_GROUPS):
        m = jnp.max(gk, axis=0, keepdims=True)
        gmask = jnp.logical_or(gmask, giota == (7 - (m & 7)))
        gk = jnp.where(gk == m, _IMIN, gk)

    sm = jnp.where(gmask, kb, zkb)

    idx_rows = []
    w_rows = []
    for _ in range(_TOP_K):
        m = jnp.max(sm, axis=(0, 1), keepdims=True)
        rm = sm == m
        w_rows.append(jnp.sum(jnp.where(rm, s3, 0.0), axis=(0, 1)).reshape(1, tb))
        idx_rows.append((63 - (m & 63)).reshape(1, tb))
        sm = jnp.where(rm, _IMIN, sm)
    wts = jnp.concatenate(w_rows, axis=0)
    inds = jnp.concatenate(idx_rows, axis=0)

    wts = wts / (jnp.sum(wts, axis=0, keepdims=True) + 1e-20) * _SCALE
    wout_ref[...] = wts.T
    iout_ref[...] = inds.T


def kernel(x_TD, kernel_DE, bias_E):
    x_TD = jnp.asarray(x_TD, jnp.float32)
    t, d = x_TD.shape
    wt = kernel_DE.astype(jnp.float32).T
    bias = bias_E.reshape(_GROUPS, _GROUP_SIZE).astype(jnp.float32)
    grid = t // _TB
    weights, indices = pl.pallas_call(
        _router_body,
        grid=(grid,),
        in_specs=[
            pl.BlockSpec((_TB, d), lambda i: (i, 0)),
            pl.BlockSpec((_E, d), lambda i: (0, 0)),
            pl.BlockSpec((_GROUPS, _GROUP_SIZE), lambda i: (0, 0)),
        ],
        out_specs=[
            pl.BlockSpec((_TB, _TOP_K), lambda i: (i, 0)),
            pl.BlockSpec((_TB, _TOP_K), lambda i: (i, 0)),
        ],
        out_shape=[
            jax.ShapeDtypeStruct((t, _TOP_K), jnp.float32),
            jax.ShapeDtypeStruct((t, _TOP_K), jnp.int32),
        ],
    )(x_TD, wt, bias)
    return (weights, indices)


# speedup vs baseline: 2.7116x; 1.0740x over previous
import jax
import jax.numpy as jnp
from jax import lax
from jax.experimental import pallas as pl

_E = 64
_GROUPS = 8
_GROUP_SIZE = 8
_TOPK_GROUPS = 4
_TOP_K = 8
_SCALE = 2.5
_TB = 1024

_IMIN = -(2 ** 31)
_FIX = float(2 ** 23)


def _router_body(x_ref, wt_ref, b_ref, wout_ref, iout_ref):
    x = x_ref[...]
    wt = wt_ref[...]
    scores = lax.dot_general(
        wt, x, (((1,), (1,)), ((), ())), preferred_element_type=jnp.float32
    )
    tb = scores.shape[-1]
    s3 = jax.nn.sigmoid(scores).reshape(_GROUPS, _GROUP_SIZE, tb)
    sb3 = s3 + b_ref[...][:, :, None]

    gio = lax.broadcasted_iota(jnp.int32, (_GROUPS, _GROUP_SIZE, tb), 0)
    mio = lax.broadcasted_iota(jnp.int32, (_GROUPS, _GROUP_SIZE, tb), 1)
    eidx = gio * _GROUP_SIZE + mio
    zkb = 63 - eidx

    q = jnp.clip(sb3, -3.9, 3.9) * _FIX
    kb = (q.astype(jnp.int32) << 6) | zkb

    m1 = jnp.max(kb, axis=1, keepdims=True)
    m2 = jnp.max(jnp.where(kb == m1, _IMIN, kb), axis=1, keepdims=True)
    gq = (m1 >> 6) + (m2 >> 6)

    giota = gio[:, :1, :]
    gk = (gq << 3) | (7 - giota)
    gmask = jnp.zeros((_GROUPS, 1, tb), jnp.bool_)
    for _ in range(_TOPK_GROUPS):
        m = jnp.max(gk, axis=0, keepdims=True)
        gmask = jnp.logical_or(gmask, giota == (7 - (m & 7)))
        gk = jnp.where(gk == m, _IMIN, gk)

    sm = jnp.where(gmask, kb, zkb)

    idx_rows = []
    w_rows = []
    for _ in range(_TOP_K):
        m = jnp.max(sm, axis=(0, 1), keepdims=True)
        rm = sm == m
        w_rows.append(jnp.sum(jnp.where(rm, s3, 0.0), axis=(0, 1)).reshape(1, tb))
        idx_rows.append((63 - (m & 63)).reshape(1, tb))
        sm = jnp.where(rm, _IMIN, sm)
    wts = jnp.concatenate(w_rows, axis=0)
    inds = jnp.concatenate(idx_rows, axis=0)

    wts = wts / (jnp.sum(wts, axis=0, keepdims=True) + 1e-20) * _SCALE
    wout_ref[...] = wts.T
    iout_ref[...] = inds.T


def kernel(x_TD, kernel_DE, bias_E):
    x_TD = jnp.asarray(x_TD, jnp.float32)
    t, d = x_TD.shape
    wt = kernel_DE.astype(jnp.float32).T
    bias = bias_E.reshape(_GROUPS, _GROUP_SIZE).astype(jnp.float32)
    grid = t // _TB
    weights, indices = pl.pallas_call(
        _router_body,
        grid=(grid,),
        in_specs=[
            pl.BlockSpec((_TB, d), lambda i: (i, 0)),
            pl.BlockSpec((_E, d), lambda i: (0, 0)),
            pl.BlockSpec((_GROUPS, _GROUP_SIZE), lambda i: (0, 0)),
        ],
        out_specs=[
            pl.BlockSpec((_TB, _TOP_K), lambda i: (i, 0)),
            pl.BlockSpec((_TB, _TOP_K), lambda i: (i, 0)),
        ],
        out_shape=[
            jax.ShapeDtypeStruct((t, _TOP_K), jnp.float32),
            jax.ShapeDtypeStruct((t, _TOP_K), jnp.int32),
        ],
    )(x_TD, wt, bias)
    return (weights, indices)
